# baseline (speedup 1.0000x reference)
import jax
import jax.numpy as jnp
from jax import lax
from jax.experimental import pallas as pl

_TI = 16
_NT = 4

_SEQ_OUT = [(0, 0), (1, 0), (0, 1), (2, 0), (0, 2), (2, 1), (0, 3), (3, 0),
            (1, 1), (3, 1), (1, 2), (3, 2), (1, 3), (2, 2), (2, 3), (3, 3)]
_SEQ_IN = [(0, 0), (0, 0), (0, 1), (0, 1), (0, 2), (0, 2), (0, 3), (0, 3),
           (1, 1), (1, 1), (1, 2), (1, 2), (1, 3), (2, 2), (2, 3), (3, 3)]


def _pack(vals):
    acc = 0
    for k, v in enumerate(vals):
        acc |= v << (2 * k)
    return acc


_TI_OUT = _pack([t[0] for t in _SEQ_OUT])
_TJ_OUT = _pack([t[1] for t in _SEQ_OUT])
_TI_IN = _pack([t[0] for t in _SEQ_IN])
_TJ_IN = _pack([t[1] for t in _SEQ_IN])


def _unpack(packed, s):
    shifted = lax.shift_right_logical(jnp.uint32(packed), (2 * s).astype(jnp.uint32))
    return (shifted & jnp.uint32(3)).astype(jnp.int32)


def _fused_kernel(adj_ref, uadj_ref, alpha_ref, uops_ref, adj_out_ref, ops_out_ref):
    s = pl.program_id(0)
    ti = _unpack(_TI_OUT, s)
    tj = _unpack(_TJ_OUT, s)

    a = adj_ref[...]
    t = -jnp.log(uadj_ref[...])
    act = 1.0 / (1.0 + t * jnp.exp(-a))
    i = ti * _TI + lax.broadcasted_iota(jnp.int32, a.shape, 0)
    j = tj * _TI + lax.broadcasted_iota(jnp.int32, a.shape, 1)
    adj_out_ref[...] = jnp.where(j > i, act, 0.0)

    @pl.when(s % 4 == 3)
    def _ops():
        e = jnp.exp(alpha_ref[...]) / (-jnp.log(uops_ref[...]))
        ops_out_ref[...] = e / jnp.sum(e, axis=1, keepdims=True)


def kernel(adj_para, ops_alpha, u_adj, u_ops):
    B, N, _ = adj_para.shape
    OPS = ops_alpha.shape[-1]

    adj_t = jnp.transpose(adj_para, (1, 2, 0))
    uadj_t = jnp.transpose(u_adj, (1, 2, 0))
    alpha_t = jnp.transpose(ops_alpha, (1, 2, 0))
    uops_t = jnp.transpose(u_ops, (1, 2, 0))

    adj_in_spec = pl.BlockSpec(
        (_TI, _TI, B), lambda s: (_unpack(_TI_IN, s), _unpack(_TJ_IN, s), 0))
    adj_out_spec = pl.BlockSpec(
        (_TI, _TI, B), lambda s: (_unpack(_TI_OUT, s), _unpack(_TJ_OUT, s), 0))
    ops_spec = pl.BlockSpec((_TI, OPS, B), lambda s: (s // 4, 0, 0))

    adj_out_t, ops_out_t = pl.pallas_call(
        _fused_kernel,
        grid=(_NT * _NT,),
        in_specs=[adj_in_spec, adj_in_spec, ops_spec, ops_spec],
        out_specs=[adj_out_spec, ops_spec],
        out_shape=[
            jax.ShapeDtypeStruct((N, N, B), adj_para.dtype),
            jax.ShapeDtypeStruct((N, OPS, B), ops_alpha.dtype),
        ],
    )(adj_t, uadj_t, alpha_t, uops_t)

    return (jnp.transpose(adj_out_t, (2, 0, 1)),
            jnp.transpose(ops_out_t, (2, 0, 1)))


# speedup vs baseline: 1.1629x; 1.1629x over previous
import jax
import jax.numpy as jnp
from jax import lax
from jax.experimental import pallas as pl

_TI = 16


def _fused_kernel(adj_ref, uadj_ref, alpha_ref, uops_ref, adj_out_ref, ops_out_ref):
    ti = pl.program_id(0)
    tj = pl.program_id(1)
    nt = pl.num_programs(1)

    a = adj_ref[...]
    t = -jnp.log(uadj_ref[...])
    act = 1.0 / (1.0 + t * jnp.exp(-a))
    i = ti * _TI + lax.broadcasted_iota(jnp.int32, a.shape, 0)
    j = tj * _TI + lax.broadcasted_iota(jnp.int32, a.shape, 1)
    adj_out_ref[...] = jnp.where(j > i, act, 0.0)

    @pl.when(tj == nt - 1)
    def _ops():
        e = jnp.exp(alpha_ref[...]) / (-jnp.log(uops_ref[...]))
        ops_out_ref[...] = e / jnp.sum(e, axis=1, keepdims=True)


def kernel(adj_para, ops_alpha, u_adj, u_ops):
    B, N, _ = adj_para.shape
    OPS = ops_alpha.shape[-1]

    adj_t = jnp.transpose(adj_para, (1, 2, 0))
    uadj_t = jnp.transpose(u_adj, (1, 2, 0))
    alpha_t = jnp.transpose(ops_alpha, (1, 2, 0))
    uops_t = jnp.transpose(u_ops, (1, 2, 0))

    nt = N // _TI
    adj_in_spec = pl.BlockSpec((_TI, _TI, B), lambda ti, tj: (ti, jnp.maximum(tj, ti), 0))
    adj_out_spec = pl.BlockSpec((_TI, _TI, B), lambda ti, tj: (ti, tj, 0))
    ops_spec = pl.BlockSpec((_TI, OPS, B), lambda ti, tj: (ti, 0, 0))

    adj_out_t, ops_out_t = pl.pallas_call(
        _fused_kernel,
        grid=(nt, nt),
        in_specs=[adj_in_spec, adj_in_spec, ops_spec, ops_spec],
        out_specs=[adj_out_spec, ops_spec],
        out_shape=[
            jax.ShapeDtypeStruct((N, N, B), adj_para.dtype),
            jax.ShapeDtypeStruct((N, OPS, B), ops_alpha.dtype),
        ],
    )(adj_t, uadj_t, alpha_t, uops_t)

    return (jnp.transpose(adj_out_t, (2, 0, 1)),
            jnp.transpose(ops_out_t, (2, 0, 1)))
